# baseline (speedup 1.0000x reference)
import functools
import math

import jax
import jax.numpy as jnp
from jax import lax
from jax.experimental import pallas as pl
from jax.experimental.pallas import tpu as pltpu
from jax.experimental.pallas import tpu_sc as plsc

N = 16384
C = 128
NV = N // 16
COLS_PER_TILE = C // 32
EPS = 1e-06
MU = (N + 1) / 2.0
VAR = (N * N - 1) / 12.0
DENOM = math.sqrt(VAR + EPS) * math.sqrt(VAR + EPS) + EPS
SCALE = 1.0 / (C * N * DENOM)

SHIFTS = (0, 11, 22)
BITS = (11, 11, 10)
SIZES = tuple(1 << b for b in BITS)
HB = max(SIZES)


def _iota16():
    return lax.iota(jnp.int32, 16)


def _transform(b):
    s = lax.shift_right_arithmetic(b, 31)
    return b ^ (s | jnp.int32(-2147483648))


def _digit(t, p):
    d = lax.shift_right_logical(t, jnp.int32(SHIFTS[p]))
    return lax.bitwise_and(d, jnp.int32(SIZES[p] - 1))


def _hist_clear(hist, sz):
    z = jnp.zeros((16,), jnp.int32)

    def body(i, _):
        for u in range(4):
            hist[pl.ds(i * 64 + u * 16, 16)] = z
        return 0

    lax.fori_loop(0, sz // 64, body, 0)


def _count_sweep(p, load_k, hist, pdest):
    def body(i, _):
        d = _digit(load_k(i), p)
        g = plsc.load_gather(hist, [d])
        cnt, is_last = plsc.scan_count(d)
        pdest[pl.ds(i * 16, 16)] = g + cnt - 1
        plsc.addupdate_scatter(hist, [d], cnt, mask=is_last)
        return 0

    lax.fori_loop(0, NV, body, 0)


def _hist_scan(p, hist):
    def body(i, tot):
        h = hist[pl.ds(i * 16, 16)]
        cs = plsc.cumsum(h)
        hist[pl.ds(i * 16, 16)] = cs - h + tot
        return tot + jnp.sum(h)

    lax.fori_loop(0, SIZES[p] // 16, body, jnp.int32(0))


def _permute(p, hist, pdest, load_kv, emit):
    def body(i, carry):
        kk, v = load_kv(i)
        d = _digit(kk, p)
        o = plsc.load_gather(hist, [d])
        dest = o + pdest[pl.ds(i * 16, 16)]
        return emit(dest, kk, v, carry)

    return lax.fori_loop(0, NV, body, jnp.zeros((16,), jnp.float32))


def _pass(p, load_k, load_kv, emit, hist, pdest):
    _hist_clear(hist, SIZES[p])
    _count_sweep(p, load_k, hist, pdest)
    _hist_scan(p, hist)
    return _permute(p, hist, pdest, load_kv, emit)


def _rank_column(raw, keyy, valy, valx, hist, pdest, final_emit):
    keyx = raw
    iota = _iota16()

    def loadk1(i):
        return _transform(raw[pl.ds(i * 16, 16)])

    def load1(i):
        return _transform(raw[pl.ds(i * 16, 16)]), i * 16 + iota

    def emit1(dest, kk, v, carry):
        plsc.store_scatter(keyy, [dest], kk)
        plsc.store_scatter(valy, [dest], v)
        return carry

    _pass(0, loadk1, load1, emit1, hist, pdest)

    def loadk2(i):
        return keyy[pl.ds(i * 16, 16)]

    def load2(i):
        return keyy[pl.ds(i * 16, 16)], valy[pl.ds(i * 16, 16)]

    def emit2(dest, kk, v, carry):
        plsc.store_scatter(keyx, [dest], kk)
        plsc.store_scatter(valx, [dest], v)
        return carry

    _pass(1, loadk2, load2, emit2, hist, pdest)

    def loadk3(i):
        return keyx[pl.ds(i * 16, 16)]

    def load3(i):
        return keyx[pl.ds(i * 16, 16)], valx[pl.ds(i * 16, 16)]

    return _pass(2, loadk3, load3, final_emit, hist, pdest)


mesh = plsc.VectorSubcoreMesh(core_axis_name="c", subcore_axis_name="s")


@functools.partial(
    pl.kernel,
    mesh=mesh,
    compiler_params=pltpu.CompilerParams(needs_layout_passes=False),
    out_type=jax.ShapeDtypeStruct((32, 16), jnp.float32),
    scratch_types=[
        pltpu.VMEM((N,), jnp.int32),
        pltpu.VMEM((N,), jnp.int32),
        pltpu.VMEM((N,), jnp.int32),
        pltpu.VMEM((N,), jnp.int32),
        pltpu.VMEM((N,), jnp.float32),
        pltpu.VMEM((N,), jnp.int32),
        pltpu.VMEM((HB,), jnp.int32),
        pltpu.VMEM((16,), jnp.float32),
    ],
)
def _sc_spearman(pred_hbm, tgt_hbm, out_hbm, raw, keyy, valy, valx, rt, pdest,
                 hist, accb):
    wid = lax.axis_index("s") * 2 + lax.axis_index("c")

    def col_body(j, acc):
        col = wid * COLS_PER_TILE + j

        pltpu.sync_copy(tgt_hbm.at[col], raw)

        def emit_t(dest, kk, v, carry):
            rank = (dest + 1).astype(jnp.float32)
            plsc.store_scatter(rt, [v], rank)
            return carry

        _rank_column(raw, keyy, valy, valx, hist, pdest, emit_t)

        pltpu.sync_copy(pred_hbm.at[col], raw)

        def emit_p(dest, kk, v, carry):
            rp = (dest + 1).astype(jnp.float32)
            g = plsc.load_gather(rt, [v])
            return carry + (rp - MU) * (g - MU)

        part = _rank_column(raw, keyy, valy, valx, hist, pdest, emit_p)
        return acc + part

    acc = lax.fori_loop(0, COLS_PER_TILE, col_body, jnp.zeros((16,), jnp.float32))
    accb[...] = acc
    pltpu.sync_copy(accb, out_hbm.at[wid])


def kernel(pred, target):
    pred_i = lax.bitcast_convert_type(pred.T, jnp.int32)
    tgt_i = lax.bitcast_convert_type(target.T, jnp.int32)
    partial = _sc_spearman(pred_i, tgt_i)
    return (-jnp.sum(partial) * jnp.float32(SCALE)).astype(jnp.float32)


# speedup vs baseline: 1.1429x; 1.1429x over previous
import functools
import math

import jax
import jax.numpy as jnp
from jax import lax
from jax.experimental import pallas as pl
from jax.experimental.pallas import tpu as pltpu
from jax.experimental.pallas import tpu_sc as plsc

N = 16384
C = 128
NV = N // 16
COLS_PER_TILE = C // 32
EPS = 1e-06
MU = (N + 1) / 2.0
VAR = (N * N - 1) / 12.0
DENOM = math.sqrt(VAR + EPS) * math.sqrt(VAR + EPS) + EPS
SCALE = 1.0 / (C * N * DENOM)

SHIFTS = (0, 11, 22)
BITS = (11, 11, 10)
SIZES = tuple(1 << b for b in BITS)
BASES = (0, SIZES[0], SIZES[0] + SIZES[1])
HTOT = sum(SIZES)


def _iota16():
    return lax.iota(jnp.int32, 16)


def _transform(b):
    s = lax.shift_right_arithmetic(b, 31)
    return b ^ (s | jnp.int32(-2147483648))


def _digit(t, p):
    if SHIFTS[p] == 0:
        return lax.bitwise_and(t, jnp.int32(SIZES[p] - 1))
    d = lax.shift_right_logical(t, jnp.int32(SHIFTS[p]))
    if SHIFTS[p] + BITS[p] >= 32:
        return d
    return lax.bitwise_and(d, jnp.int32(SIZES[p] - 1))


def _hist_clear(hist):
    z = jnp.zeros((16,), jnp.int32)

    def body(i, _):
        for u in range(4):
            hist[pl.ds(i * 64 + u * 16, 16)] = z
        return 0

    lax.fori_loop(0, HTOT // 64, body, 0)


def _hist_sweep(raw, hist):
    def body(i, _):
        t = _transform(raw[pl.ds(i * 16, 16)])
        for p in range(3):
            d = _digit(t, p) + jnp.int32(BASES[p])
            cnt, is_last = plsc.scan_count(d)
            plsc.addupdate_scatter(hist, [d], cnt, mask=is_last)
        return 0

    lax.fori_loop(0, NV, body, 0)


def _hist_scan(hist):
    for p in range(3):
        base = BASES[p]

        def body(i, tot, base=base):
            h = hist[pl.ds(base + i * 16, 16)]
            cs = plsc.cumsum(h)
            hist[pl.ds(base + i * 16, 16)] = cs - h + tot
            return tot + jnp.sum(h)

        lax.fori_loop(0, SIZES[p] // 16, body, jnp.int32(0))


def _permute(p, hist, load_kv, emit, dig=None):
    base = jnp.int32(BASES[p])
    dfn = _digit if dig is None else dig

    def body(i, carry):
        k, v = load_kv(i)
        d = dfn(k, p) + base
        g = plsc.load_gather(hist, [d])
        cnt, is_last = plsc.scan_count(d)
        dest = g + cnt - 1
        carry = emit(dest, k, v, carry)
        plsc.addupdate_scatter(hist, [d], cnt, mask=is_last)
        return carry

    return lax.fori_loop(0, NV, body, jnp.zeros((16,), jnp.float32))


def _rank_column(raw, keyy, valy, hist, rt, final_emit):
    keyx = raw

    _hist_clear(hist)
    _hist_sweep(raw, hist)
    _hist_scan(hist)

    iota = _iota16()

    def load1(i):
        return _transform(raw[pl.ds(i * 16, 16)]), i * 16 + iota

    def emit1(dest, k, v, carry):
        plsc.store_scatter(keyy, [dest], k)
        plsc.store_scatter(valy, [dest], v)
        return carry

    _permute(0, hist, load1, emit1)

    def load2(i):
        return keyy[pl.ds(i * 16, 16)], valy[pl.ds(i * 16, 16)]

    def emit2(dest, k, v, carry):
        pk = lax.bitwise_or(
            lax.shift_left(lax.shift_right_logical(k, 22), jnp.int32(14)), v)
        plsc.store_scatter(keyx, [dest], pk)
        return carry

    _permute(1, hist, load2, emit2)

    def load3(i):
        pk = keyx[pl.ds(i * 16, 16)]
        return pk, lax.bitwise_and(pk, jnp.int32(16383))

    def dig3(k, p):
        return lax.shift_right_logical(k, jnp.int32(14))

    return _permute(2, hist, load3, final_emit, dig=dig3)


mesh = plsc.VectorSubcoreMesh(core_axis_name="c", subcore_axis_name="s")


@functools.partial(
    pl.kernel,
    mesh=mesh,
    compiler_params=pltpu.CompilerParams(needs_layout_passes=False),
    out_type=jax.ShapeDtypeStruct((32, 16), jnp.float32),
    scratch_types=[
        pltpu.VMEM((N,), jnp.int32),
        pltpu.VMEM((N,), jnp.int32),
        pltpu.VMEM((N,), jnp.int32),
        pltpu.VMEM((N,), jnp.float32),
        pltpu.VMEM((HTOT,), jnp.int32),
        pltpu.VMEM((16,), jnp.float32),
    ],
)
def _sc_spearman(pred_hbm, tgt_hbm, out_hbm, raw, keyy, valy, rt, hist, accb):
    wid = lax.axis_index("s") * 2 + lax.axis_index("c")

    def col_body(j, acc):
        col = wid * COLS_PER_TILE + j

        pltpu.sync_copy(tgt_hbm.at[col], raw)

        def emit_t(dest, k, v, carry):
            rc = dest.astype(jnp.float32) - jnp.float32(MU - 1.0)
            plsc.store_scatter(rt, [v], rc)
            return carry

        _rank_column(raw, keyy, valy, hist, rt, emit_t)

        pltpu.sync_copy(pred_hbm.at[col], raw)

        def emit_p(dest, k, v, carry):
            rpc = dest.astype(jnp.float32) - jnp.float32(MU - 1.0)
            g = plsc.load_gather(rt, [v])
            return carry + rpc * g

        part = _rank_column(raw, keyy, valy, hist, rt, emit_p)
        return acc + part

    acc = lax.fori_loop(0, COLS_PER_TILE, col_body, jnp.zeros((16,), jnp.float32))
    accb[...] = acc
    pltpu.sync_copy(accb, out_hbm.at[wid])


def kernel(pred, target):
    pred_i = lax.bitcast_convert_type(pred.T, jnp.int32)
    tgt_i = lax.bitcast_convert_type(target.T, jnp.int32)
    partial = _sc_spearman(pred_i, tgt_i)
    return (-jnp.sum(partial) * jnp.float32(SCALE)).astype(jnp.float32)


# speedup vs baseline: 1.3949x; 1.2205x over previous
import functools
import math

import jax
import jax.numpy as jnp
from jax import lax
from jax.experimental import pallas as pl
from jax.experimental.pallas import tpu as pltpu
from jax.experimental.pallas import tpu_sc as plsc

N = 16384
C = 128
NV = N // 16
COLS_PER_TILE = C // 32
EPS = 1e-06
MU = (N + 1) / 2.0
VAR = (N * N - 1) / 12.0
DENOM = math.sqrt(VAR + EPS) * math.sqrt(VAR + EPS) + EPS
SCALE = 1.0 / (C * N * DENOM)

SHIFTS = (0, 11, 22)
BITS = (11, 11, 10)
SIZES = tuple(1 << b for b in BITS)
BASES = (0, SIZES[0], SIZES[0] + SIZES[1])
HTOT = sum(SIZES)


def _iota16():
    return lax.iota(jnp.int32, 16)


def _transform(b):
    s = lax.shift_right_arithmetic(b, 31)
    return b ^ (s | jnp.int32(-2147483648))


def _digit(t, p):
    if SHIFTS[p] == 0:
        return lax.bitwise_and(t, jnp.int32(SIZES[p] - 1))
    d = lax.shift_right_logical(t, jnp.int32(SHIFTS[p]))
    if SHIFTS[p] + BITS[p] >= 32:
        return d
    return lax.bitwise_and(d, jnp.int32(SIZES[p] - 1))


def _hist_clear(hist):
    z = jnp.zeros((16,), jnp.int32)

    def body(i, _):
        for u in range(4):
            hist[pl.ds(i * 64 + u * 16, 16)] = z
        return 0

    lax.fori_loop(0, HTOT // 64, body, 0)


def _hist_sweep(raw, hist):
    ones = jnp.ones((16,), jnp.int32)

    def body(i, _):
        t = _transform(raw[pl.ds(i * 16, 16)])
        for p in range(3):
            d = _digit(t, p) + jnp.int32(BASES[p])
            plsc.addupdate_scatter(hist, [d], ones)
        return 0

    lax.fori_loop(0, NV, body, 0)


def _hist_scan(hist):
    for p in range(3):
        base = BASES[p]

        def body(i, tot, base=base):
            h = hist[pl.ds(base + i * 16, 16)]
            cs = plsc.cumsum(h)
            hist[pl.ds(base + i * 16, 16)] = cs - h + tot
            return tot + jnp.sum(h)

        lax.fori_loop(0, SIZES[p] // 16, body, jnp.int32(0))


def _permute(p, hist, load_kv, emit, dig=None):
    base = jnp.int32(BASES[p])
    dfn = _digit if dig is None else dig

    ones = jnp.ones((16,), jnp.int32)

    def body(i, carry):
        k, v = load_kv(i)
        d = dfn(k, p) + base
        g = plsc.load_gather(hist, [d])
        plsc.addupdate_scatter(hist, [d], ones)
        cnt, _ = plsc.scan_count(d)
        dest = g + cnt - 1
        return emit(dest, k, v, carry)

    return lax.fori_loop(0, NV, body, jnp.zeros((16,), jnp.float32))


def _rank_column(raw, keyy, valy, hist, rt, final_emit):
    keyx = raw

    _hist_clear(hist)
    _hist_sweep(raw, hist)
    _hist_scan(hist)

    iota = _iota16()

    def load1(i):
        return _transform(raw[pl.ds(i * 16, 16)]), i * 16 + iota

    def emit1(dest, k, v, carry):
        plsc.store_scatter(keyy, [dest], k)
        plsc.store_scatter(valy, [dest], v)
        return carry

    _permute(0, hist, load1, emit1)

    def load2(i):
        return keyy[pl.ds(i * 16, 16)], valy[pl.ds(i * 16, 16)]

    def emit2(dest, k, v, carry):
        pk = lax.bitwise_or(
            lax.shift_left(lax.shift_right_logical(k, 22), jnp.int32(14)), v)
        plsc.store_scatter(keyx, [dest], pk)
        return carry

    _permute(1, hist, load2, emit2)

    def load3(i):
        pk = keyx[pl.ds(i * 16, 16)]
        return pk, lax.bitwise_and(pk, jnp.int32(16383))

    def dig3(k, p):
        return lax.shift_right_logical(k, jnp.int32(14))

    return _permute(2, hist, load3, final_emit, dig=dig3)


mesh = plsc.VectorSubcoreMesh(core_axis_name="c", subcore_axis_name="s")


@functools.partial(
    pl.kernel,
    mesh=mesh,
    compiler_params=pltpu.CompilerParams(needs_layout_passes=False),
    out_type=jax.ShapeDtypeStruct((32, 16), jnp.float32),
    scratch_types=[
        pltpu.VMEM((N,), jnp.int32),
        pltpu.VMEM((N,), jnp.int32),
        pltpu.VMEM((N,), jnp.int32),
        pltpu.VMEM((N,), jnp.float32),
        pltpu.VMEM((HTOT,), jnp.int32),
        pltpu.VMEM((16,), jnp.float32),
    ],
)
def _sc_spearman(pred_hbm, tgt_hbm, out_hbm, raw, keyy, valy, rt, hist, accb):
    wid = lax.axis_index("s") * 2 + lax.axis_index("c")

    def col_body(j, acc):
        col = wid * COLS_PER_TILE + j

        pltpu.sync_copy(tgt_hbm.at[col], raw)

        def emit_t(dest, k, v, carry):
            rc = dest.astype(jnp.float32) - jnp.float32(MU - 1.0)
            plsc.store_scatter(rt, [v], rc)
            return carry

        _rank_column(raw, keyy, valy, hist, rt, emit_t)

        pltpu.sync_copy(pred_hbm.at[col], raw)

        def emit_p(dest, k, v, carry):
            rpc = dest.astype(jnp.float32) - jnp.float32(MU - 1.0)
            g = plsc.load_gather(rt, [v])
            return carry + rpc * g

        part = _rank_column(raw, keyy, valy, hist, rt, emit_p)
        return acc + part

    acc = lax.fori_loop(0, COLS_PER_TILE, col_body, jnp.zeros((16,), jnp.float32))
    accb[...] = acc
    pltpu.sync_copy(accb, out_hbm.at[wid])


def kernel(pred, target):
    pred_i = lax.bitcast_convert_type(pred.T, jnp.int32)
    tgt_i = lax.bitcast_convert_type(target.T, jnp.int32)
    partial = _sc_spearman(pred_i, tgt_i)
    return (-jnp.sum(partial) * jnp.float32(SCALE)).astype(jnp.float32)
